# baseline (speedup 1.0000x reference)
import functools

import jax
import jax.numpy as jnp
from jax import lax
from jax.experimental import pallas as pl
from jax.experimental.pallas import tpu as pltpu
from jax.experimental.pallas import tpu_sc as plsc

_N = 50000
_F = 512
_G = 256
_STEPS = 6
_BLK = 5000
_NBLK = _N // _BLK
_W = 32
_NW = _G // _W
_MAXP = _NBLK + _NW - 1
_NEG = -1e30

_NSC = 32
_CHUNK = 1664
_NPAD = _NSC * _CHUNK


def _phase1_kernel(cidx_ref, widx_ref, tot_ref, first_ref,
                   x_ref, ids_ref, wih_ref, whh_ref, bih_ref, bhh_ref,
                   qp_ref, hp_ref, cp_ref,
                   e_out, m_out, d_out, h_out, c_out,
                   m_s, d_s):
    k = pl.program_id(0)

    @pl.when(k == 0)
    def _lstm():
        gates = (
            lax.dot_general(qp_ref[...], wih_ref[...],
                            (((1,), (1,)), ((), ())),
                            preferred_element_type=jnp.float32)
            + lax.dot_general(hp_ref[...], whh_ref[...],
                              (((1,), (1,)), ((), ())),
                              preferred_element_type=jnp.float32)
            + bih_ref[...] + bhh_ref[...]
        )
        i_g = gates[:, :_F]
        f_g = gates[:, _F:2 * _F]
        g_g = gates[:, 2 * _F:3 * _F]
        o_g = gates[:, 3 * _F:]
        c_new = jax.nn.sigmoid(f_g) * cp_ref[...] + jax.nn.sigmoid(i_g) * jnp.tanh(g_g)
        h_new = jax.nn.sigmoid(o_g) * jnp.tanh(c_new)
        h_out[...] = h_new
        c_out[...] = c_new
        m_s[...] = jnp.full((_G, 1), _NEG, jnp.float32)
        d_s[...] = jnp.zeros((_G, 1), jnp.float32)

    w = widx_ref[k]
    base = w * _W

    @pl.when(k < tot_ref[0])
    def _pair():
        xb = x_ref[...]
        ids = ids_ref[0]
        hw = h_out[pl.ds(base, _W), :]
        et = lax.dot_general(hw, xb, (((1,), (1,)), ((), ())),
                             preferred_element_type=jnp.float32)
        lg = ids - base
        og = lax.broadcasted_iota(jnp.int32, (_W, _BLK), 0) == lg
        e_row = jnp.sum(jnp.where(og, et, 0.0), axis=0, keepdims=True)

        @pl.when(first_ref[k] == 1)
        def _():
            e_out[0] = e_row

        @pl.when(first_ref[k] == 0)
        def _():
            e_out[0] = e_out[0] + e_row

        m_blk = jnp.max(jnp.where(og, et, _NEG), axis=1, keepdims=True)
        m_old = m_s[pl.ds(base, _W), :]
        m_new = jnp.maximum(m_old, m_blk)
        alpha = jnp.exp(m_old - m_new)
        p = jnp.where(og, jnp.exp(et - m_new), 0.0)
        d_blk = jnp.sum(p, axis=1, keepdims=True)
        d_s[pl.ds(base, _W), :] = d_s[pl.ds(base, _W), :] * alpha + d_blk
        m_s[pl.ds(base, _W), :] = m_new

    @pl.when(k == _MAXP - 1)
    def _fin():
        m_out[...] = m_s[...]
        d_out[...] = d_s[...]


def _phase2_kernel(cidx_ref, widx_ref, tot_ref, first_ref,
                   x_ref, ids_ref, a_ref, h_ref,
                   qs_out, r_s):
    k = pl.program_id(0)

    @pl.when(k == 0)
    def _init():
        r_s[...] = jnp.zeros((_G, _F), jnp.float32)

    w = widx_ref[k]
    base = w * _W

    @pl.when(k < tot_ref[0])
    def _pair():
        xb = x_ref[...]
        ids = ids_ref[0]
        a_row = a_ref[0]
        lg = ids - base
        og = lax.broadcasted_iota(jnp.int32, (_W, _BLK), 0) == lg
        pw = jnp.where(og, a_row, 0.0)
        r_s[pl.ds(base, _W), :] = r_s[pl.ds(base, _W), :] + lax.dot_general(
            pw, xb, (((1,), (0,)), ((), ())), preferred_element_type=jnp.float32)

    @pl.when(k == _MAXP - 1)
    def _fin():
        qs_out[:, :_F] = h_ref[...]
        qs_out[:, _F:] = r_s[...]


def _sc_weights(e_pad, ids_pad, m, d):
    mesh = plsc.VectorSubcoreMesh(core_axis_name="c", subcore_axis_name="s")

    @functools.partial(
        pl.kernel, mesh=mesh,
        out_type=jax.ShapeDtypeStruct((_NPAD,), jnp.float32),
        scratch_types=[
            pltpu.VMEM((_CHUNK,), jnp.float32),
            pltpu.VMEM((_CHUNK,), jnp.int32),
            pltpu.VMEM((_CHUNK,), jnp.float32),
            pltpu.VMEM((_CHUNK,), jnp.float32),
            pltpu.VMEM((_CHUNK,), jnp.float32),
            pltpu.SemaphoreType.DMA,
        ],
    )
    def sc_a(e_hbm, ids_hbm, m_hbm, d_hbm, out_hbm, e_v, id_v, mg_v, dg_v, a_v, sem):
        wid = lax.axis_index("s") * 2 + lax.axis_index("c")
        base = wid * _CHUNK
        pltpu.sync_copy(e_hbm.at[pl.ds(base, _CHUNK)], e_v)
        pltpu.sync_copy(ids_hbm.at[pl.ds(base, _CHUNK)], id_v)
        for g in range(_CHUNK // 128):
            sl = pl.ds(g * 128, 128)
            pltpu.async_copy(m_hbm.at[id_v.at[sl]], mg_v.at[sl], sem).wait()
            pltpu.async_copy(d_hbm.at[id_v.at[sl]], dg_v.at[sl], sem).wait()

        def body(j, carry):
            sl = pl.ds(j * 16, 16)
            a_v[sl] = jnp.exp(e_v[sl] - mg_v[sl]) / (dg_v[sl] + 1e-16)
            return carry

        lax.fori_loop(0, _CHUNK // 16, body, 0)
        pltpu.sync_copy(a_v, out_hbm.at[pl.ds(base, _CHUNK)])

    return sc_a(e_pad, ids_pad, m, d)


@jax.jit
def _run(x, ids3, ids_pad, cidx, widx, tot, first, w_ih, w_hh, b_ih2, b_hh2):
    grid_spec1 = pltpu.PrefetchScalarGridSpec(
        num_scalar_prefetch=4,
        grid=(_MAXP,),
        in_specs=[
            pl.BlockSpec((_BLK, _F), lambda k, ci, wi, tt, fi: (ci[k], 0)),
            pl.BlockSpec((1, 1, _BLK), lambda k, ci, wi, tt, fi: (ci[k], 0, 0)),
            pl.BlockSpec((4 * _F, 2 * _F), lambda k, ci, wi, tt, fi: (0, 0)),
            pl.BlockSpec((4 * _F, _F), lambda k, ci, wi, tt, fi: (0, 0)),
            pl.BlockSpec((1, 4 * _F), lambda k, ci, wi, tt, fi: (0, 0)),
            pl.BlockSpec((1, 4 * _F), lambda k, ci, wi, tt, fi: (0, 0)),
            pl.BlockSpec((_G, 2 * _F), lambda k, ci, wi, tt, fi: (0, 0)),
            pl.BlockSpec((_G, _F), lambda k, ci, wi, tt, fi: (0, 0)),
            pl.BlockSpec((_G, _F), lambda k, ci, wi, tt, fi: (0, 0)),
        ],
        out_specs=[
            pl.BlockSpec((1, 1, _BLK), lambda k, ci, wi, tt, fi: (ci[k], 0, 0)),
            pl.BlockSpec((_G, 1), lambda k, ci, wi, tt, fi: (0, 0)),
            pl.BlockSpec((_G, 1), lambda k, ci, wi, tt, fi: (0, 0)),
            pl.BlockSpec((_G, _F), lambda k, ci, wi, tt, fi: (0, 0)),
            pl.BlockSpec((_G, _F), lambda k, ci, wi, tt, fi: (0, 0)),
        ],
        scratch_shapes=[
            pltpu.VMEM((_G, 1), jnp.float32),
            pltpu.VMEM((_G, 1), jnp.float32),
        ],
    )
    phase1 = pl.pallas_call(
        _phase1_kernel,
        grid_spec=grid_spec1,
        out_shape=[
            jax.ShapeDtypeStruct((_NBLK, 1, _BLK), jnp.float32),
            jax.ShapeDtypeStruct((_G, 1), jnp.float32),
            jax.ShapeDtypeStruct((_G, 1), jnp.float32),
            jax.ShapeDtypeStruct((_G, _F), jnp.float32),
            jax.ShapeDtypeStruct((_G, _F), jnp.float32),
        ],
    )
    grid_spec2 = pltpu.PrefetchScalarGridSpec(
        num_scalar_prefetch=4,
        grid=(_MAXP,),
        in_specs=[
            pl.BlockSpec((_BLK, _F), lambda k, ci, wi, tt, fi: (ci[k], 0)),
            pl.BlockSpec((1, 1, _BLK), lambda k, ci, wi, tt, fi: (ci[k], 0, 0)),
            pl.BlockSpec((1, 1, _BLK), lambda k, ci, wi, tt, fi: (ci[k], 0, 0)),
            pl.BlockSpec((_G, _F), lambda k, ci, wi, tt, fi: (0, 0)),
        ],
        out_specs=pl.BlockSpec((_G, 2 * _F), lambda k, ci, wi, tt, fi: (0, 0)),
        scratch_shapes=[pltpu.VMEM((_G, _F), jnp.float32)],
    )
    phase2 = pl.pallas_call(
        _phase2_kernel,
        grid_spec=grid_spec2,
        out_shape=jax.ShapeDtypeStruct((_G, 2 * _F), jnp.float32),
    )

    q_star = jnp.zeros((_G, 2 * _F), jnp.float32)
    h = jnp.zeros((_G, _F), jnp.float32)
    c = jnp.zeros((_G, _F), jnp.float32)
    for _ in range(_STEPS):
        e3, m, d, h, c = phase1(cidx, widx, tot, first,
                                x, ids3, w_ih, w_hh, b_ih2, b_hh2,
                                q_star, h, c)
        e_pad = jnp.zeros((_NPAD,), jnp.float32).at[:_N].set(e3.reshape(_N))
        a_pad = _sc_weights(e_pad, ids_pad, m.reshape(_G), d.reshape(_G))
        a3 = a_pad[:_N].reshape(_NBLK, 1, _BLK)
        q_star = phase2(cidx, widx, tot, first, x, ids3, a3, h)
    return q_star


def _schedule(batch):
    firsts = batch[:: _BLK]
    lasts = batch[_BLK - 1:: _BLK]
    w_lo = firsts // _W
    w_hi = lasts // _W
    cnt = w_hi - w_lo + 1
    s_off = jnp.cumsum(cnt) - cnt
    total = s_off[-1] + cnt[-1]
    k = jnp.arange(_MAXP)
    cidx = jnp.clip(jnp.searchsorted(s_off, k, side="right") - 1, 0, _NBLK - 1)
    widx = jnp.clip(w_lo[cidx] + (k - s_off[cidx]), 0, _NW - 1)
    first = jnp.concatenate([jnp.ones((1,), jnp.int32),
                             (cidx[1:] != cidx[:-1]).astype(jnp.int32)])
    return (cidx.astype(jnp.int32), widx.astype(jnp.int32),
            total.astype(jnp.int32).reshape(1), first)


def kernel(x, batch, W_ih, W_hh, b_ih, b_hh):
    batch = batch.astype(jnp.int32)
    cidx, widx, tot, first = _schedule(batch)
    ids3 = batch.reshape(_NBLK, 1, _BLK)
    ids_pad = jnp.zeros((_NPAD,), jnp.int32).at[:_N].set(batch)
    return _run(x, ids3, ids_pad, cidx, widx, tot, first, W_ih, W_hh,
                b_ih.reshape(1, -1), b_hh.reshape(1, -1))


# speedup vs baseline: 4.2953x; 4.2953x over previous
import functools

import jax
import jax.numpy as jnp
from jax import lax
from jax.experimental import pallas as pl
from jax.experimental.pallas import tpu as pltpu

_N = 50000
_F = 512
_G = 256
_STEPS = 6
_BLK = 5000
_NBLK = _N // _BLK
_W = 32
_NW = _G // _W
_MAXP = _NBLK + _NW - 1
_NEG = -1e30


def _kernel(cidx_ref, widx_ref, tot_ref,
            x_ref, ids_ref, wih_ref, whh_ref, bih_ref, bhh_ref,
            out_ref,
            h_s, c_s, q_s, m_s, d_s, r_s):
    s = pl.program_id(0)
    k = pl.program_id(1)

    @pl.when(jnp.logical_and(s == 0, k == 0))
    def _init():
        q_s[...] = jnp.zeros((_G, 2 * _F), jnp.float32)
        h_s[...] = jnp.zeros((_G, _F), jnp.float32)
        c_s[...] = jnp.zeros((_G, _F), jnp.float32)

    @pl.when(k == 0)
    def _lstm():
        gates = (
            lax.dot_general(q_s[...], wih_ref[...],
                            (((1,), (1,)), ((), ())),
                            preferred_element_type=jnp.float32)
            + lax.dot_general(h_s[...], whh_ref[...],
                              (((1,), (1,)), ((), ())),
                              preferred_element_type=jnp.float32)
            + bih_ref[...] + bhh_ref[...]
        )
        i_g = gates[:, :_F]
        f_g = gates[:, _F:2 * _F]
        g_g = gates[:, 2 * _F:3 * _F]
        o_g = gates[:, 3 * _F:]
        c_new = jax.nn.sigmoid(f_g) * c_s[...] + jax.nn.sigmoid(i_g) * jnp.tanh(g_g)
        h_new = jax.nn.sigmoid(o_g) * jnp.tanh(c_new)
        h_s[...] = h_new
        c_s[...] = c_new
        m_s[...] = jnp.full((_G, 1), _NEG, jnp.float32)
        d_s[...] = jnp.zeros((_G, 1), jnp.float32)
        r_s[...] = jnp.zeros((_G, _F), jnp.float32)

    w = widx_ref[k]
    base = w * _W

    @pl.when(k < tot_ref[0])
    def _pair():
        xb = x_ref[...]
        ids = ids_ref[0]
        hw = h_s[pl.ds(base, _W), :]
        et = lax.dot_general(hw, xb, (((1,), (1,)), ((), ())),
                             preferred_element_type=jnp.float32)
        lg = ids - base
        og = lax.broadcasted_iota(jnp.int32, (_W, _BLK), 0) == lg
        m_blk = jnp.max(jnp.where(og, et, _NEG), axis=1, keepdims=True)
        m_old = m_s[pl.ds(base, _W), :]
        m_new = jnp.maximum(m_old, m_blk)
        alpha = jnp.exp(m_old - m_new)
        p = jnp.where(og, jnp.exp(et - m_new), 0.0)
        d_blk = jnp.sum(p, axis=1, keepdims=True)
        r_s[pl.ds(base, _W), :] = r_s[pl.ds(base, _W), :] * alpha + lax.dot_general(
            p, xb, (((1,), (0,)), ((), ())), preferred_element_type=jnp.float32)
        d_s[pl.ds(base, _W), :] = d_s[pl.ds(base, _W), :] * alpha + d_blk
        m_s[pl.ds(base, _W), :] = m_new

    @pl.when(k == _MAXP - 1)
    def _finish():
        r = r_s[...] / (d_s[...] + 1e-16)
        q_s[:, :_F] = h_s[...]
        q_s[:, _F:] = r

    @pl.when(jnp.logical_and(s == _STEPS - 1, k == _MAXP - 1))
    def _emit():
        out_ref[...] = q_s[...]


@jax.jit
def _run(x, ids3, cidx, widx, tot, w_ih, w_hh, b_ih2, b_hh2):
    grid_spec = pltpu.PrefetchScalarGridSpec(
        num_scalar_prefetch=3,
        grid=(_STEPS, _MAXP),
        in_specs=[
            pl.BlockSpec((_BLK, _F), lambda s, k, ci, wi, tt: (ci[k], 0)),
            pl.BlockSpec((1, 1, _BLK), lambda s, k, ci, wi, tt: (ci[k], 0, 0)),
            pl.BlockSpec((4 * _F, 2 * _F), lambda s, k, ci, wi, tt: (0, 0)),
            pl.BlockSpec((4 * _F, _F), lambda s, k, ci, wi, tt: (0, 0)),
            pl.BlockSpec((1, 4 * _F), lambda s, k, ci, wi, tt: (0, 0)),
            pl.BlockSpec((1, 4 * _F), lambda s, k, ci, wi, tt: (0, 0)),
        ],
        out_specs=pl.BlockSpec((_G, 2 * _F), lambda s, k, ci, wi, tt: (0, 0)),
        scratch_shapes=[
            pltpu.VMEM((_G, _F), jnp.float32),
            pltpu.VMEM((_G, _F), jnp.float32),
            pltpu.VMEM((_G, 2 * _F), jnp.float32),
            pltpu.VMEM((_G, 1), jnp.float32),
            pltpu.VMEM((_G, 1), jnp.float32),
            pltpu.VMEM((_G, _F), jnp.float32),
        ],
    )
    return pl.pallas_call(
        _kernel,
        grid_spec=grid_spec,
        out_shape=jax.ShapeDtypeStruct((_G, 2 * _F), jnp.float32),
    )(cidx, widx, tot, x, ids3, w_ih, w_hh, b_ih2, b_hh2)


def _schedule(batch):
    firsts = batch[:: _BLK]
    lasts = batch[_BLK - 1:: _BLK]
    w_lo = firsts // _W
    w_hi = lasts // _W
    cnt = w_hi - w_lo + 1
    s_off = jnp.cumsum(cnt) - cnt
    total = s_off[-1] + cnt[-1]
    k = jnp.arange(_MAXP)
    cidx = jnp.clip(jnp.searchsorted(s_off, k, side="right") - 1, 0, _NBLK - 1)
    widx = jnp.clip(w_lo[cidx] + (k - s_off[cidx]), 0, _NW - 1)
    return (cidx.astype(jnp.int32), widx.astype(jnp.int32),
            total.astype(jnp.int32).reshape(1))


def kernel(x, batch, W_ih, W_hh, b_ih, b_hh):
    batch = batch.astype(jnp.int32)
    cidx, widx, tot = _schedule(batch)
    ids3 = batch.reshape(_NBLK, 1, _BLK)
    return _run(x, ids3, cidx, widx, tot, W_ih, W_hh,
                b_ih.reshape(1, -1), b_hh.reshape(1, -1))


# speedup vs baseline: 5.2239x; 1.2162x over previous
import functools

import jax
import jax.numpy as jnp
from jax import lax
from jax.experimental import pallas as pl
from jax.experimental.pallas import tpu as pltpu

_N = 50000
_F = 512
_G = 256
_STEPS = 6
_BLK = 5000
_NBLK = _N // _BLK
_W = 64
_NW = _G // _W
_MAXP = _NBLK + _NW - 1
_NEG = -1e30


def _kernel(cidx_ref, widx_ref, tot_ref,
            x_ref, ids_ref, wih_ref, whh_ref, bih_ref, bhh_ref,
            out_ref,
            h_s, c_s, q_s, m_s, d_s, r_s):
    s = pl.program_id(0)
    k = pl.program_id(1)

    @pl.when(jnp.logical_and(s == 0, k == 0))
    def _init():
        q_s[...] = jnp.zeros((_G, 2 * _F), jnp.float32)
        h_s[...] = jnp.zeros((_G, _F), jnp.float32)
        c_s[...] = jnp.zeros((_G, _F), jnp.float32)

    @pl.when(k == 0)
    def _lstm():
        gates = (
            lax.dot_general(q_s[...], wih_ref[...],
                            (((1,), (1,)), ((), ())),
                            preferred_element_type=jnp.float32)
            + lax.dot_general(h_s[...], whh_ref[...],
                              (((1,), (1,)), ((), ())),
                              preferred_element_type=jnp.float32)
            + bih_ref[...] + bhh_ref[...]
        )
        i_g = gates[:, :_F]
        f_g = gates[:, _F:2 * _F]
        g_g = gates[:, 2 * _F:3 * _F]
        o_g = gates[:, 3 * _F:]
        c_new = jax.nn.sigmoid(f_g) * c_s[...] + jax.nn.sigmoid(i_g) * jnp.tanh(g_g)
        h_new = jax.nn.sigmoid(o_g) * jnp.tanh(c_new)
        h_s[...] = h_new
        c_s[...] = c_new
        m_s[...] = jnp.full((_G, 1), _NEG, jnp.float32)
        d_s[...] = jnp.zeros((_G, 1), jnp.float32)
        r_s[...] = jnp.zeros((_G, _F), jnp.float32)

    w = widx_ref[k]
    base = w * _W

    @pl.when(k < tot_ref[0])
    def _pair():
        xb = x_ref[...]
        ids = ids_ref[0]
        hw = h_s[pl.ds(base, _W), :]
        et = lax.dot_general(hw, xb, (((1,), (1,)), ((), ())),
                             preferred_element_type=jnp.float32)
        lg = ids - base
        og = lax.broadcasted_iota(jnp.int32, (_W, _BLK), 0) == lg
        m_blk = jnp.max(jnp.where(og, et, _NEG), axis=1, keepdims=True)
        m_old = m_s[pl.ds(base, _W), :]
        m_new = jnp.maximum(m_old, m_blk)
        alpha = jnp.exp(m_old - m_new)
        p = jnp.where(og, jnp.exp(et - m_new), 0.0)
        d_blk = jnp.sum(p, axis=1, keepdims=True)
        r_s[pl.ds(base, _W), :] = r_s[pl.ds(base, _W), :] * alpha + lax.dot_general(
            p, xb, (((1,), (0,)), ((), ())), preferred_element_type=jnp.float32)
        d_s[pl.ds(base, _W), :] = d_s[pl.ds(base, _W), :] * alpha + d_blk
        m_s[pl.ds(base, _W), :] = m_new

    @pl.when(k == _MAXP - 1)
    def _finish():
        r = r_s[...] / (d_s[...] + 1e-16)
        q_s[:, :_F] = h_s[...]
        q_s[:, _F:] = r

    @pl.when(jnp.logical_and(s == _STEPS - 1, k == _MAXP - 1))
    def _emit():
        out_ref[...] = q_s[...]


@jax.jit
def _run(x, ids3, cidx, widx, tot, w_ih, w_hh, b_ih2, b_hh2):
    grid_spec = pltpu.PrefetchScalarGridSpec(
        num_scalar_prefetch=3,
        grid=(_STEPS, _MAXP),
        in_specs=[
            pl.BlockSpec((_BLK, _F), lambda s, k, ci, wi, tt: (ci[k], 0)),
            pl.BlockSpec((1, 1, _BLK), lambda s, k, ci, wi, tt: (ci[k], 0, 0)),
            pl.BlockSpec((4 * _F, 2 * _F), lambda s, k, ci, wi, tt: (0, 0)),
            pl.BlockSpec((4 * _F, _F), lambda s, k, ci, wi, tt: (0, 0)),
            pl.BlockSpec((1, 4 * _F), lambda s, k, ci, wi, tt: (0, 0)),
            pl.BlockSpec((1, 4 * _F), lambda s, k, ci, wi, tt: (0, 0)),
        ],
        out_specs=pl.BlockSpec((_G, 2 * _F), lambda s, k, ci, wi, tt: (0, 0)),
        scratch_shapes=[
            pltpu.VMEM((_G, _F), jnp.float32),
            pltpu.VMEM((_G, _F), jnp.float32),
            pltpu.VMEM((_G, 2 * _F), jnp.float32),
            pltpu.VMEM((_G, 1), jnp.float32),
            pltpu.VMEM((_G, 1), jnp.float32),
            pltpu.VMEM((_G, _F), jnp.float32),
        ],
    )
    return pl.pallas_call(
        _kernel,
        grid_spec=grid_spec,
        out_shape=jax.ShapeDtypeStruct((_G, 2 * _F), jnp.float32),
    )(cidx, widx, tot, x, ids3, w_ih, w_hh, b_ih2, b_hh2)


def _schedule(batch):
    firsts = batch[:: _BLK]
    lasts = batch[_BLK - 1:: _BLK]
    w_lo = firsts // _W
    w_hi = lasts // _W
    cnt = w_hi - w_lo + 1
    s_off = jnp.cumsum(cnt) - cnt
    total = s_off[-1] + cnt[-1]
    k = jnp.arange(_MAXP)
    cidx = jnp.clip(jnp.searchsorted(s_off, k, side="right") - 1, 0, _NBLK - 1)
    widx = jnp.clip(w_lo[cidx] + (k - s_off[cidx]), 0, _NW - 1)
    return (cidx.astype(jnp.int32), widx.astype(jnp.int32),
            total.astype(jnp.int32).reshape(1))


def kernel(x, batch, W_ih, W_hh, b_ih, b_hh):
    batch = batch.astype(jnp.int32)
    cidx, widx, tot = _schedule(batch)
    ids3 = batch.reshape(_NBLK, 1, _BLK)
    return _run(x, ids3, cidx, widx, tot, W_ih, W_hh,
                b_ih.reshape(1, -1), b_hh.reshape(1, -1))


# speedup vs baseline: 5.7121x; 1.0935x over previous
import functools

import jax
import jax.numpy as jnp
from jax import lax
from jax.experimental import pallas as pl
from jax.experimental.pallas import tpu as pltpu

_N = 50000
_F = 512
_G = 256
_STEPS = 6
_BLK = 5000
_NBLK = _N // _BLK
_W = 128
_NW = _G // _W
_MAXP = _NBLK + _NW - 1
_NEG = -1e30


def _kernel(cidx_ref, widx_ref, tot_ref,
            x_ref, ids_ref, wih_ref, whh_ref, bih_ref, bhh_ref,
            out_ref,
            h_s, c_s, q_s, m_s, d_s, r_s):
    s = pl.program_id(0)
    k = pl.program_id(1)

    @pl.when(jnp.logical_and(s == 0, k == 0))
    def _init():
        q_s[...] = jnp.zeros((_G, 2 * _F), jnp.float32)
        h_s[...] = jnp.zeros((_G, _F), jnp.float32)
        c_s[...] = jnp.zeros((_G, _F), jnp.float32)

    @pl.when(k == 0)
    def _lstm():
        gates = (
            lax.dot_general(q_s[...], wih_ref[...],
                            (((1,), (1,)), ((), ())),
                            preferred_element_type=jnp.float32)
            + lax.dot_general(h_s[...], whh_ref[...],
                              (((1,), (1,)), ((), ())),
                              preferred_element_type=jnp.float32)
            + bih_ref[...] + bhh_ref[...]
        )
        i_g = gates[:, :_F]
        f_g = gates[:, _F:2 * _F]
        g_g = gates[:, 2 * _F:3 * _F]
        o_g = gates[:, 3 * _F:]
        c_new = jax.nn.sigmoid(f_g) * c_s[...] + jax.nn.sigmoid(i_g) * jnp.tanh(g_g)
        h_new = jax.nn.sigmoid(o_g) * jnp.tanh(c_new)
        h_s[...] = h_new
        c_s[...] = c_new
        m_s[...] = jnp.full((_G, 1), _NEG, jnp.float32)
        d_s[...] = jnp.zeros((_G, 1), jnp.float32)
        r_s[...] = jnp.zeros((_G, _F), jnp.float32)

    w = widx_ref[k]
    base = w * _W

    @pl.when(k < tot_ref[0])
    def _pair():
        xb = x_ref[...]
        ids = ids_ref[0]
        hw = h_s[pl.ds(base, _W), :]
        et = lax.dot_general(hw, xb, (((1,), (1,)), ((), ())),
                             preferred_element_type=jnp.float32)
        lg = ids - base
        og = lax.broadcasted_iota(jnp.int32, (_W, _BLK), 0) == lg
        m_blk = jnp.max(jnp.where(og, et, _NEG), axis=1, keepdims=True)
        m_old = m_s[pl.ds(base, _W), :]
        m_new = jnp.maximum(m_old, m_blk)
        alpha = jnp.exp(m_old - m_new)
        p = jnp.where(og, jnp.exp(et - m_new), 0.0)
        d_blk = jnp.sum(p, axis=1, keepdims=True)
        r_s[pl.ds(base, _W), :] = r_s[pl.ds(base, _W), :] * alpha + lax.dot_general(
            p, xb, (((1,), (0,)), ((), ())), preferred_element_type=jnp.float32)
        d_s[pl.ds(base, _W), :] = d_s[pl.ds(base, _W), :] * alpha + d_blk
        m_s[pl.ds(base, _W), :] = m_new

    @pl.when(k == _MAXP - 1)
    def _finish():
        r = r_s[...] / (d_s[...] + 1e-16)
        q_s[:, :_F] = h_s[...]
        q_s[:, _F:] = r

    @pl.when(jnp.logical_and(s == _STEPS - 1, k == _MAXP - 1))
    def _emit():
        out_ref[...] = q_s[...]


@jax.jit
def _run(x, ids3, cidx, widx, tot, w_ih, w_hh, b_ih2, b_hh2):
    grid_spec = pltpu.PrefetchScalarGridSpec(
        num_scalar_prefetch=3,
        grid=(_STEPS, _MAXP),
        in_specs=[
            pl.BlockSpec((_BLK, _F), lambda s, k, ci, wi, tt: (ci[k], 0)),
            pl.BlockSpec((1, 1, _BLK), lambda s, k, ci, wi, tt: (ci[k], 0, 0)),
            pl.BlockSpec((4 * _F, 2 * _F), lambda s, k, ci, wi, tt: (0, 0)),
            pl.BlockSpec((4 * _F, _F), lambda s, k, ci, wi, tt: (0, 0)),
            pl.BlockSpec((1, 4 * _F), lambda s, k, ci, wi, tt: (0, 0)),
            pl.BlockSpec((1, 4 * _F), lambda s, k, ci, wi, tt: (0, 0)),
        ],
        out_specs=pl.BlockSpec((_G, 2 * _F), lambda s, k, ci, wi, tt: (0, 0)),
        scratch_shapes=[
            pltpu.VMEM((_G, _F), jnp.float32),
            pltpu.VMEM((_G, _F), jnp.float32),
            pltpu.VMEM((_G, 2 * _F), jnp.float32),
            pltpu.VMEM((_G, 1), jnp.float32),
            pltpu.VMEM((_G, 1), jnp.float32),
            pltpu.VMEM((_G, _F), jnp.float32),
        ],
    )
    return pl.pallas_call(
        _kernel,
        grid_spec=grid_spec,
        out_shape=jax.ShapeDtypeStruct((_G, 2 * _F), jnp.float32),
    )(cidx, widx, tot, x, ids3, w_ih, w_hh, b_ih2, b_hh2)


def _schedule(batch):
    firsts = batch[:: _BLK]
    lasts = batch[_BLK - 1:: _BLK]
    w_lo = firsts // _W
    w_hi = lasts // _W
    cnt = w_hi - w_lo + 1
    s_off = jnp.cumsum(cnt) - cnt
    total = s_off[-1] + cnt[-1]
    k = jnp.arange(_MAXP)
    cidx = jnp.clip(jnp.searchsorted(s_off, k, side="right") - 1, 0, _NBLK - 1)
    widx = jnp.clip(w_lo[cidx] + (k - s_off[cidx]), 0, _NW - 1)
    return (cidx.astype(jnp.int32), widx.astype(jnp.int32),
            total.astype(jnp.int32).reshape(1))


def kernel(x, batch, W_ih, W_hh, b_ih, b_hh):
    batch = batch.astype(jnp.int32)
    cidx, widx, tot = _schedule(batch)
    ids3 = batch.reshape(_NBLK, 1, _BLK)
    return _run(x, ids3, cidx, widx, tot, W_ih, W_hh,
                b_ih.reshape(1, -1), b_hh.reshape(1, -1))
